# baseline (speedup 1.0000x reference)
import functools

import jax
import jax.numpy as jnp
from jax.experimental import pallas as pl
from jax.experimental.pallas import tpu as pltpu
from jax.experimental.pallas import tpu_sc as plsc

BSZ = 16
_BE = 2000
_SC_NC = 2
_SC_NS = 16
_SC_CH = 200


def _rbf(v, centers):
    gammap = ((10.0 - 0.0) / 2.0) ** 2
    return jnp.exp(-((v[:, None] - centers[None, :]) ** 2) / gammap)


def _relu(v):
    return jnp.maximum(v, 0.0)


def _dot(a, b):
    return jnp.dot(a, b, preferred_element_type=jnp.float32)


def _sc_gather_pair(tblr, tblc, row, col):
    e = row.shape[0]
    nw = _SC_NC * _SC_NS
    per_w = e // nw
    n_ch = per_w // _SC_CH
    dr = tblr.shape[1]
    dc = tblc.shape[1]
    dtr = tblr.dtype
    dtc = tblc.dtype
    mesh = plsc.VectorSubcoreMesh(core_axis_name="c", subcore_axis_name="s",
                                  num_cores=_SC_NC, num_subcores=_SC_NS)

    def body(tblr_hbm, tblc_hbm, row_hbm, col_hbm, g_hbm, gd_hbm,
             ri_v, ci_v, gb_v, gdb_v, sem_r, sem_c):
        wid = jax.lax.axis_index("s") * _SC_NC + jax.lax.axis_index("c")
        base = wid * per_w
        for j in range(n_ch):
            off = base + j * _SC_CH
            pltpu.sync_copy(row_hbm.at[pl.ds(off, _SC_CH)], ri_v)
            pltpu.sync_copy(col_hbm.at[pl.ds(off, _SC_CH)], ci_v)
            a = pltpu.async_copy(tblr_hbm.at[ri_v], gb_v, sem_r)
            b = pltpu.async_copy(tblc_hbm.at[ci_v], gdb_v, sem_c)
            a.wait()
            b.wait()
            pltpu.sync_copy(gb_v, g_hbm.at[pl.ds(off, _SC_CH)])
            pltpu.sync_copy(gdb_v, gd_hbm.at[pl.ds(off, _SC_CH)])

    f = pl.kernel(
        body,
        out_type=[jax.ShapeDtypeStruct((e, dr), dtr),
                  jax.ShapeDtypeStruct((e, dc), dtc)],
        mesh=mesh,
        scratch_types=[
            pltpu.VMEM((_SC_CH,), jnp.int32),
            pltpu.VMEM((_SC_CH,), jnp.int32),
            pltpu.VMEM((_SC_CH, dr), dtr),
            pltpu.VMEM((_SC_CH, dc), dtc),
            pltpu.SemaphoreType.DMA,
            pltpu.SemaphoreType.DMA,
        ],
    )
    return f(tblr, tblc, row, col)


def _sc_scatter_add(vals, idx, n):
    e, w = vals.shape
    nw = _SC_NC * _SC_NS
    per_w = e // nw
    n_ch = per_w // _SC_CH
    n2 = ((n + 16 * 8 - 1) // (16 * 8)) * (16 * 8)
    rps = n2 // _SC_NS
    mesh = plsc.VectorSubcoreMesh(core_axis_name="c", subcore_axis_name="s",
                                  num_cores=_SC_NC, num_subcores=_SC_NS)

    def body(vals_hbm, idx_hbm, zeros_hbm, out_hbm, vb, iv, acc_sh):
        c = jax.lax.axis_index("c")
        s = jax.lax.axis_index("s")
        wid = s * _SC_NC + c
        pltpu.sync_copy(zeros_hbm.at[pl.ds(s * rps, rps)],
                        acc_sh.at[pl.ds(s * rps, rps)])
        plsc.subcore_barrier()
        base = wid * per_w
        for j in range(n_ch):
            off = base + j * _SC_CH
            pltpu.sync_copy(idx_hbm.at[pl.ds(off, _SC_CH)], iv)
            pltpu.sync_copy(vals_hbm.at[pl.ds(off, _SC_CH)], vb)
            pltpu.sync_copy(vb, acc_sh.at[iv], add=True)
        plsc.subcore_barrier()
        pltpu.sync_copy(acc_sh.at[pl.ds(s * rps, rps)],
                        out_hbm.at[pl.ds(c * n2 + s * rps, rps)])

    f = pl.kernel(
        body,
        out_type=jax.ShapeDtypeStruct((2 * n2, w), jnp.float32),
        mesh=mesh,
        scratch_types=[
            pltpu.VMEM((_SC_CH, w), jnp.float32),
            pltpu.VMEM((_SC_CH,), jnp.int32),
            pltpu.VMEM_SHARED((n2, w), jnp.float32),
        ],
    )
    out = f(vals, idx, jnp.zeros((n2, w), jnp.float32))
    return out[:n] + out[n2:n2 + n]


def _edge_l1_body(g_ref, gd_ref, eidx_ref, t2_ref, w2t_ref, b2_ref,
                  wn1et_ref, bn1_ref, wn2t_ref, bn2_ref, ea_ref, n1_ref):
    d = gd_ref.shape[1]
    g = g_ref[...].astype(jnp.float32)
    gd = gd_ref[...].astype(jnp.float32)
    eidx = eidx_ref[0, 0, :]
    sel = jnp.where(eidx[:, None] == 0, t2_ref[0:1, :], t2_ref[1:2, :])
    h1 = _relu(g[:, :d] + gd + sel)
    ea = h1 + _relu(_dot(h1, w2t_ref[...]) + b2_ref[...])
    n1h = _relu(g[:, d:] + _dot(ea, wn1et_ref[...]) + bn1_ref[...])
    n1 = n1h + _relu(_dot(n1h, wn2t_ref[...]) + bn2_ref[...])
    ea_ref[...] = ea
    n1_ref[...] = n1


def _edge_l2_body(g_ref, gd_ref, ea_ref, w1et_ref, b1_ref, w2t_ref, b2_ref,
                  wn1et_ref, bn1_ref, wn2t_ref, bn2_ref, n1_ref):
    d = gd_ref.shape[1]
    g = g_ref[...].astype(jnp.float32)
    gd = gd_ref[...].astype(jnp.float32)
    h1 = _relu(g[:, :d] + gd
               + _dot(ea_ref[...], w1et_ref[...]) + b1_ref[...])
    ea = h1 + _relu(_dot(h1, w2t_ref[...]) + b2_ref[...])
    n1h = _relu(g[:, d:] + _dot(ea, wn1et_ref[...]) + bn1_ref[...])
    n1 = n1h + _relu(_dot(n1h, wn2t_ref[...]) + bn2_ref[...])
    n1_ref[...] = n1


def _edge_block_spec(be, d):
    return pl.BlockSpec((be, d), lambda i: (i, 0))


def _const_spec(shape):
    nd = len(shape)
    return pl.BlockSpec(shape, lambda i: (0,) * nd)


def _run_edge_l1(g, gd, eidx3, t2, w2t, b2, wn1et, bn1, wn2t, bn2, be):
    ep, d = gd.shape
    grid = ep // be
    out_shape = [jax.ShapeDtypeStruct((ep, d), jnp.float32),
                 jax.ShapeDtypeStruct((ep, d), jnp.float32)]
    return pl.pallas_call(
        _edge_l1_body,
        grid=(grid,),
        in_specs=[
            _edge_block_spec(be, 2 * d),
            _edge_block_spec(be, d),
            pl.BlockSpec((1, 1, be), lambda i: (i, 0, 0)),
            _const_spec((2, d)),
            _const_spec((d, d)), _const_spec((1, d)),
            _const_spec((d, d)), _const_spec((1, d)),
            _const_spec((d, d)), _const_spec((1, d)),
        ],
        out_specs=[_edge_block_spec(be, d), _edge_block_spec(be, d)],
        out_shape=out_shape,
        compiler_params=pltpu.CompilerParams(
            dimension_semantics=("arbitrary",)),
    )(g, gd, eidx3, t2, w2t, b2, wn1et, bn1, wn2t, bn2)


def _run_edge_l2(g, gd, ea, w1et, b1, w2t, b2, wn1et, bn1, wn2t, bn2, be):
    ep, d = gd.shape
    grid = ep // be
    return pl.pallas_call(
        _edge_l2_body,
        grid=(grid,),
        in_specs=[
            _edge_block_spec(be, 2 * d),
            _edge_block_spec(be, d),
            _edge_block_spec(be, d),
            _const_spec((d, d)), _const_spec((1, d)),
            _const_spec((d, d)), _const_spec((1, d)),
            _const_spec((d, d)), _const_spec((1, d)),
            _const_spec((d, d)), _const_spec((1, d)),
        ],
        out_specs=[_edge_block_spec(be, d)],
        out_shape=[jax.ShapeDtypeStruct((ep, d), jnp.float32)],
        compiler_params=pltpu.CompilerParams(
            dimension_semantics=("arbitrary",)),
    )(g, gd, ea, w1et, b1, w2t, b2, wn1et, bn1, wn2t, bn2)[0]


def kernel(x_fp, x_feature, params, centers, edge_index, edge_attr_idx,
           batch, num_graphs):
    n = x_fp.shape[0]
    e = edge_index.shape[1]
    d = params["edge_embed"].shape[1]

    mol = x_feature[:, 0] == 0.0
    rec = x_feature[:, 0] == 1.0
    fp = _dot(x_fp, params["fp_W"].T)
    r3 = _rbf(x_feature[:, 3], centers)
    r4 = _rbf(x_feature[:, 4], centers)
    first = jnp.where(mol[:, None], fp, jnp.where(rec[:, None], r3, 0.0))
    second = jnp.where((mol | rec)[:, None], r4, 0.0)
    x = jnp.concatenate([first, second], axis=1)
    u = jnp.zeros((BSZ, d), jnp.float32) + jnp.asarray(num_graphs, jnp.float32) * 0.0

    row = edge_index[0]
    col = edge_index[1]

    be = min(_BE, e)
    ep = ((e + be - 1) // be) * be
    pad = ep - e
    row_p = jnp.concatenate([row, jnp.zeros((pad,), row.dtype)]) if pad else row
    col_scat = jnp.concatenate([col, jnp.full((pad,), n, col.dtype)]) if pad else col
    col_p = jnp.concatenate([col, jnp.zeros((pad,), col.dtype)]) if pad else col
    eidx_p = (jnp.concatenate([edge_attr_idx, jnp.zeros((pad,), edge_attr_idx.dtype)])
              if pad else edge_attr_idx)
    eidx3 = eidx_p.astype(jnp.int32).reshape(ep // be, 1, be)

    use_sc = (pad == 0) and e % (_SC_NC * _SC_NS * _SC_CH) == 0

    deg = jax.ops.segment_sum(jnp.ones((ep,), jnp.float32), col_scat,
                              num_segments=n + 1)[:n]
    deg = jnp.maximum(deg, 1.0)[:, None]

    onehot_g = (batch[None, :] == jnp.arange(BSZ, dtype=batch.dtype)[:, None]
                ).astype(jnp.float32)
    gcount = jnp.maximum(jnp.sum(onehot_g, axis=1), 1.0)[:, None]

    for li, lp in enumerate(params["layers"]):
        w1 = lp["edge"][0]["W"]
        b1 = lp["edge"][0]["b"]
        w2 = lp["edge"][1]["W"]
        b2 = lp["edge"][1]["b"]
        wn1 = lp["node1"][0]["W"]
        bn1 = lp["node1"][0]["b"]
        wn2 = lp["node1"][1]["W"]
        bn2 = lp["node1"][1]["b"]

        psu = _dot(x, w1[:, :d].T) + _dot(u, w1[:, 3 * d:].T)[batch]
        pd = _dot(x, w1[:, d:2 * d].T)
        px = _dot(x, wn1[:, :d].T)

        if use_sc:
            tblr = jnp.concatenate([psu, px], axis=1).astype(jnp.bfloat16)
            tblr_i = jax.lax.bitcast_convert_type(
                tblr.reshape(n, d, 2), jnp.int32)
            g_i, gd = _sc_gather_pair(tblr_i, pd, row, col)
            g = jax.lax.bitcast_convert_type(g_i, jnp.bfloat16).reshape(e, 2 * d)
        else:
            g = jnp.concatenate([psu[row_p], px[row_p]], axis=1)
            gd = pd[col_p]

        if li == 0:
            t2 = _dot(params["edge_embed"], w1[:, 2 * d:3 * d].T) + b1[None, :]
            ea, n1 = _run_edge_l1(
                g, gd, eidx3, t2,
                w2.T, b2[None, :], wn1[:, d:].T, bn1[None, :],
                wn2.T, bn2[None, :], be)
        else:
            n1 = _run_edge_l2(
                g, gd, ea,
                w1[:, 2 * d:3 * d].T, b1[None, :],
                w2.T, b2[None, :], wn1[:, d:].T, bn1[None, :],
                wn2.T, bn2[None, :], be)

        if use_sc:
            agg = _sc_scatter_add(n1, col, n) / deg
        else:
            agg = jax.ops.segment_sum(n1, col_scat, num_segments=n + 1)[:n] / deg

        wa = lp["node2"][0]["W"]
        ba = lp["node2"][0]["b"]
        wb = lp["node2"][1]["W"]
        bb = lp["node2"][1]["b"]
        xh = _relu(_dot(x, wa[:, :d].T) + _dot(agg, wa[:, d:2 * d].T)
                   + _dot(u, wa[:, 2 * d:].T)[batch] + ba[None, :])
        x = xh + _relu(_dot(xh, wb.T) + bb[None, :])

        if li + 1 < len(params["layers"]):
            gmean = _dot(onehot_g, x) / gcount
            wg = lp["global"][0]["W"]
            bg = lp["global"][0]["b"]
            wg2 = lp["global"][1]["W"]
            bg2 = lp["global"][1]["b"]
            uh = _relu(_dot(u, wg[:, :d].T) + _dot(gmean, wg[:, d:].T)
                       + bg[None, :])
            u = uh + _relu(_dot(uh, wg2.T) + bg2[None, :])

    return _dot(x, params["out"]["W"].T) + params["out"]["b"][None, :]


# speedup vs baseline: 1.6171x; 1.6171x over previous
import functools

import jax
import jax.numpy as jnp
from jax.experimental import pallas as pl
from jax.experimental.pallas import tpu as pltpu
from jax.experimental.pallas import tpu_sc as plsc

BSZ = 16
_BE = 2000
_SC_NC = 2
_SC_NS = 16
_SC_CH = 200


def _pick_ch(per_w):
    for ch in (256, 200, 128, 104, 64, 40, 8):
        if per_w % ch == 0:
            return ch
    return None


def _rbf(v, centers):
    gammap = ((10.0 - 0.0) / 2.0) ** 2
    return jnp.exp(-((v[:, None] - centers[None, :]) ** 2) / gammap)


def _relu(v):
    return jnp.maximum(v, 0.0)


def _dot(a, b):
    return jnp.dot(a, b, preferred_element_type=jnp.float32)


def _sc_gather_pair(tblr, tblc, row, col):
    e = row.shape[0]
    nw = _SC_NC * _SC_NS
    per_w = e // nw
    ch = _pick_ch(per_w)
    n_ch = per_w // ch
    dr = tblr.shape[1]
    dc = tblc.shape[1]
    dtr = tblr.dtype
    dtc = tblc.dtype
    mesh = plsc.VectorSubcoreMesh(core_axis_name="c", subcore_axis_name="s",
                                  num_cores=_SC_NC, num_subcores=_SC_NS)

    def body(tblr_hbm, tblc_hbm, row_hbm, col_hbm, g_hbm, gd_hbm,
             ri_v, ci_v, gb_v, gdb_v, sem_r, sem_c):
        wid = jax.lax.axis_index("s") * _SC_NC + jax.lax.axis_index("c")
        base = wid * per_w
        for j in range(n_ch):
            off = base + j * ch
            pltpu.sync_copy(row_hbm.at[pl.ds(off, ch)], ri_v)
            pltpu.sync_copy(col_hbm.at[pl.ds(off, ch)], ci_v)
            a = pltpu.async_copy(tblr_hbm.at[ri_v], gb_v, sem_r)
            b = pltpu.async_copy(tblc_hbm.at[ci_v], gdb_v, sem_c)
            a.wait()
            b.wait()
            pltpu.sync_copy(gb_v, g_hbm.at[pl.ds(off, ch)])
            pltpu.sync_copy(gdb_v, gd_hbm.at[pl.ds(off, ch)])

    f = pl.kernel(
        body,
        out_type=[jax.ShapeDtypeStruct((e, dr), dtr),
                  jax.ShapeDtypeStruct((e, dc), dtc)],
        mesh=mesh,
        scratch_types=[
            pltpu.VMEM((ch,), jnp.int32),
            pltpu.VMEM((ch,), jnp.int32),
            pltpu.VMEM((ch, dr), dtr),
            pltpu.VMEM((ch, dc), dtc),
            pltpu.SemaphoreType.DMA,
            pltpu.SemaphoreType.DMA,
        ],
    )
    return f(tblr, tblc, row, col)


def _sc_scatter_add(vals, idx, n):
    e, w = vals.shape
    nw = _SC_NC * _SC_NS
    per_w = e // nw
    ch = _pick_ch(per_w)
    n_ch = per_w // ch
    n2 = ((n + 16 * 8 - 1) // (16 * 8)) * (16 * 8)
    rps = n2 // _SC_NS
    mesh = plsc.VectorSubcoreMesh(core_axis_name="c", subcore_axis_name="s",
                                  num_cores=_SC_NC, num_subcores=_SC_NS)

    def body(vals_hbm, idx_hbm, zeros_hbm, out_hbm, vb, iv, acc_sh):
        c = jax.lax.axis_index("c")
        s = jax.lax.axis_index("s")
        wid = s * _SC_NC + c
        pltpu.sync_copy(zeros_hbm.at[pl.ds(s * rps, rps)],
                        acc_sh.at[pl.ds(s * rps, rps)])
        plsc.subcore_barrier()
        base = wid * per_w
        for j in range(n_ch):
            off = base + j * ch
            pltpu.sync_copy(idx_hbm.at[pl.ds(off, ch)], iv)
            pltpu.sync_copy(vals_hbm.at[pl.ds(off, ch)], vb)
            pltpu.sync_copy(vb, acc_sh.at[iv], add=True)
        plsc.subcore_barrier()
        pltpu.sync_copy(acc_sh.at[pl.ds(s * rps, rps)],
                        out_hbm.at[pl.ds(c * n2 + s * rps, rps)])

    f = pl.kernel(
        body,
        out_type=jax.ShapeDtypeStruct((2 * n2, w), jnp.float32),
        mesh=mesh,
        scratch_types=[
            pltpu.VMEM((ch, w), jnp.float32),
            pltpu.VMEM((ch,), jnp.int32),
            pltpu.VMEM_SHARED((n2, w), jnp.float32),
        ],
    )
    out = f(vals, idx, jnp.zeros((n2, w), jnp.float32))
    return out[:n] + out[n2:n2 + n]


def _edge_l1_body(g_ref, gd_ref, eidx_ref, t2_ref, w2t_ref, b2_ref,
                  wn1et_ref, bn1_ref, wn2t_ref, bn2_ref, ea_ref, n1_ref):
    d = gd_ref.shape[1]
    g = g_ref[...].astype(jnp.float32)
    gd = gd_ref[...].astype(jnp.float32)
    eidx = eidx_ref[0, 0, :]
    sel = jnp.where(eidx[:, None] == 0, t2_ref[0:1, :], t2_ref[1:2, :])
    h1 = _relu(g[:, :d] + gd + sel)
    ea = h1 + _relu(_dot(h1, w2t_ref[...]) + b2_ref[...])
    n1h = _relu(g[:, d:] + _dot(ea, wn1et_ref[...]) + bn1_ref[...])
    n1 = n1h + _relu(_dot(n1h, wn2t_ref[...]) + bn2_ref[...])
    ea_ref[...] = ea
    n1_ref[...] = n1


def _edge_l2_body(g_ref, gd_ref, ea_ref, w1et_ref, b1_ref, w2t_ref, b2_ref,
                  wn1et_ref, bn1_ref, wn2t_ref, bn2_ref, n1_ref):
    d = gd_ref.shape[1]
    g = g_ref[...].astype(jnp.float32)
    gd = gd_ref[...].astype(jnp.float32)
    h1 = _relu(g[:, :d] + gd
               + _dot(ea_ref[...], w1et_ref[...]) + b1_ref[...])
    ea = h1 + _relu(_dot(h1, w2t_ref[...]) + b2_ref[...])
    n1h = _relu(g[:, d:] + _dot(ea, wn1et_ref[...]) + bn1_ref[...])
    n1 = n1h + _relu(_dot(n1h, wn2t_ref[...]) + bn2_ref[...])
    n1_ref[...] = n1


def _edge_block_spec(be, d):
    return pl.BlockSpec((be, d), lambda i: (i, 0))


def _const_spec(shape):
    nd = len(shape)
    return pl.BlockSpec(shape, lambda i: (0,) * nd)


def _run_edge_l1(g, gd, eidx3, t2, w2t, b2, wn1et, bn1, wn2t, bn2, be):
    ep, d = gd.shape
    grid = ep // be
    out_shape = [jax.ShapeDtypeStruct((ep, d), jnp.float32),
                 jax.ShapeDtypeStruct((ep, d), jnp.float32)]
    return pl.pallas_call(
        _edge_l1_body,
        grid=(grid,),
        in_specs=[
            _edge_block_spec(be, 2 * d),
            _edge_block_spec(be, d),
            pl.BlockSpec((1, 1, be), lambda i: (i, 0, 0)),
            _const_spec((2, d)),
            _const_spec((d, d)), _const_spec((1, d)),
            _const_spec((d, d)), _const_spec((1, d)),
            _const_spec((d, d)), _const_spec((1, d)),
        ],
        out_specs=[_edge_block_spec(be, d), _edge_block_spec(be, d)],
        out_shape=out_shape,
        compiler_params=pltpu.CompilerParams(
            dimension_semantics=("arbitrary",)),
    )(g, gd, eidx3, t2, w2t, b2, wn1et, bn1, wn2t, bn2)


def _run_edge_l2(g, gd, ea, w1et, b1, w2t, b2, wn1et, bn1, wn2t, bn2, be):
    ep, d = gd.shape
    grid = ep // be
    return pl.pallas_call(
        _edge_l2_body,
        grid=(grid,),
        in_specs=[
            _edge_block_spec(be, 2 * d),
            _edge_block_spec(be, d),
            _edge_block_spec(be, d),
            _const_spec((d, d)), _const_spec((1, d)),
            _const_spec((d, d)), _const_spec((1, d)),
            _const_spec((d, d)), _const_spec((1, d)),
            _const_spec((d, d)), _const_spec((1, d)),
        ],
        out_specs=[_edge_block_spec(be, d)],
        out_shape=[jax.ShapeDtypeStruct((ep, d), jnp.float32)],
        compiler_params=pltpu.CompilerParams(
            dimension_semantics=("arbitrary",)),
    )(g, gd, ea, w1et, b1, w2t, b2, wn1et, bn1, wn2t, bn2)[0]


def kernel(x_fp, x_feature, params, centers, edge_index, edge_attr_idx,
           batch, num_graphs):
    n = x_fp.shape[0]
    e = edge_index.shape[1]
    d = params["edge_embed"].shape[1]

    mol = x_feature[:, 0] == 0.0
    rec = x_feature[:, 0] == 1.0
    fp = _dot(x_fp, params["fp_W"].T)
    r3 = _rbf(x_feature[:, 3], centers)
    r4 = _rbf(x_feature[:, 4], centers)
    first = jnp.where(mol[:, None], fp, jnp.where(rec[:, None], r3, 0.0))
    second = jnp.where((mol | rec)[:, None], r4, 0.0)
    x = jnp.concatenate([first, second], axis=1)
    u = jnp.zeros((BSZ, d), jnp.float32) + jnp.asarray(num_graphs, jnp.float32) * 0.0

    row = edge_index[0]
    col = edge_index[1]

    be = min(_BE, e)
    ep = ((e + be - 1) // be) * be
    pad = ep - e
    row_p = jnp.concatenate([row, jnp.zeros((pad,), row.dtype)]) if pad else row
    col_scat = jnp.concatenate([col, jnp.full((pad,), n, col.dtype)]) if pad else col
    col_p = jnp.concatenate([col, jnp.zeros((pad,), col.dtype)]) if pad else col
    eidx_p = (jnp.concatenate([edge_attr_idx, jnp.zeros((pad,), edge_attr_idx.dtype)])
              if pad else edge_attr_idx)
    eidx3 = eidx_p.astype(jnp.int32).reshape(ep // be, 1, be)

    use_sc = (pad == 0) and e % (_SC_NC * _SC_NS * _SC_CH) == 0

    if use_sc:
        h = e // 2
        unit = _SC_NC * _SC_NS * 256
        hp = ((h + unit - 1) // unit) * unit
        hpad = hp - h
        be2 = 2048 if hp % 2048 == 0 else be

        def _mkhalf(lo, hi):
            r = row[lo:hi]
            c = col[lo:hi]
            ei = edge_attr_idx[lo:hi]
            if hpad:
                r = jnp.concatenate([r, jnp.zeros((hpad,), r.dtype)])
                c_g = jnp.concatenate([c, jnp.zeros((hpad,), c.dtype)])
                c_s = jnp.concatenate([c, jnp.full((hpad,), n, c.dtype)])
                ei = jnp.concatenate([ei, jnp.zeros((hpad,), ei.dtype)])
            else:
                c_g = c_s = c
            return r, c_g, c_s, ei.astype(jnp.int32).reshape(hp // be2, 1, be2)

        halves = [_mkhalf(0, h), _mkhalf(h, e)]

    deg = jax.ops.segment_sum(jnp.ones((ep,), jnp.float32), col_scat,
                              num_segments=n + 1)[:n]
    deg = jnp.maximum(deg, 1.0)[:, None]

    onehot_g = (batch[None, :] == jnp.arange(BSZ, dtype=batch.dtype)[:, None]
                ).astype(jnp.float32)
    gcount = jnp.maximum(jnp.sum(onehot_g, axis=1), 1.0)[:, None]

    for li, lp in enumerate(params["layers"]):
        w1 = lp["edge"][0]["W"]
        b1 = lp["edge"][0]["b"]
        w2 = lp["edge"][1]["W"]
        b2 = lp["edge"][1]["b"]
        wn1 = lp["node1"][0]["W"]
        bn1 = lp["node1"][0]["b"]
        wn2 = lp["node1"][1]["W"]
        bn2 = lp["node1"][1]["b"]

        psu = _dot(x, w1[:, :d].T) + _dot(u, w1[:, 3 * d:].T)[batch]
        pd = _dot(x, w1[:, d:2 * d].T)
        px = _dot(x, wn1[:, :d].T)

        t2 = _dot(params["edge_embed"], w1[:, 2 * d:3 * d].T) + b1[None, :]
        wargs_l1 = (w2.T, b2[None, :], wn1[:, d:].T, bn1[None, :],
                    wn2.T, bn2[None, :])
        wargs_l2 = (w1[:, 2 * d:3 * d].T, b1[None, :]) + wargs_l1

        if use_sc:
            tblr = jnp.concatenate([psu, px], axis=1)
            gs = [_sc_gather_pair(tblr, pd, r, c_g)
                  for (r, c_g, c_s, ei3) in halves]
            ssum = None
            ea_new = []
            for hi, ((r, c_g, c_s, ei3), (g, gd)) in enumerate(zip(halves, gs)):
                if li == 0:
                    eah, n1 = _run_edge_l1(g, gd, ei3, t2, *wargs_l1, be2)
                    ea_new.append(eah)
                else:
                    n1 = _run_edge_l2(g, gd, ea[hi], *wargs_l2, be2)
                s = _sc_scatter_add(n1, c_s, n)
                ssum = s if ssum is None else ssum + s
            if li == 0:
                ea = ea_new
            agg = ssum / deg
        else:
            g = jnp.concatenate([psu[row_p], px[row_p]], axis=1)
            gd = pd[col_p]
            if li == 0:
                ea, n1 = _run_edge_l1(g, gd, eidx3, t2, *wargs_l1, be)
            else:
                n1 = _run_edge_l2(g, gd, ea, *wargs_l2, be)
            agg = jax.ops.segment_sum(n1, col_scat, num_segments=n + 1)[:n] / deg

        wa = lp["node2"][0]["W"]
        ba = lp["node2"][0]["b"]
        wb = lp["node2"][1]["W"]
        bb = lp["node2"][1]["b"]
        xh = _relu(_dot(x, wa[:, :d].T) + _dot(agg, wa[:, d:2 * d].T)
                   + _dot(u, wa[:, 2 * d:].T)[batch] + ba[None, :])
        x = xh + _relu(_dot(xh, wb.T) + bb[None, :])

        if li + 1 < len(params["layers"]):
            gmean = _dot(onehot_g, x) / gcount
            wg = lp["global"][0]["W"]
            bg = lp["global"][0]["b"]
            wg2 = lp["global"][1]["W"]
            bg2 = lp["global"][1]["b"]
            uh = _relu(_dot(u, wg[:, :d].T) + _dot(gmean, wg[:, d:].T)
                       + bg[None, :])
            u = uh + _relu(_dot(uh, wg2.T) + bg2[None, :])

    return _dot(x, params["out"]["W"].T) + params["out"]["b"][None, :]


# speedup vs baseline: 2.1691x; 1.3413x over previous
import functools

import jax
import jax.numpy as jnp
from jax.experimental import pallas as pl
from jax.experimental.pallas import tpu as pltpu
from jax.experimental.pallas import tpu_sc as plsc

BSZ = 16
_BE = 2000
_SC_NC = 2
_SC_NS = 16
_SC_CH = 200


def _pick_ch(per_w):
    for ch in (256, 200, 128, 104, 64, 40, 8):
        if per_w % ch == 0:
            return ch
    return None


def _rbf(v, centers):
    gammap = ((10.0 - 0.0) / 2.0) ** 2
    return jnp.exp(-((v[:, None] - centers[None, :]) ** 2) / gammap)


def _relu(v):
    return jnp.maximum(v, 0.0)


def _dot(a, b):
    return jnp.dot(a, b, preferred_element_type=jnp.float32)


def _sc_gather_pair(tblr, tblc, row, col):
    e = row.shape[0]
    nw = _SC_NC * _SC_NS
    per_w = e // nw
    ch = _pick_ch(per_w)
    n_ch = per_w // ch
    dr = tblr.shape[1]
    dc = tblc.shape[1]
    dtr = tblr.dtype
    dtc = tblc.dtype
    mesh = plsc.VectorSubcoreMesh(core_axis_name="c", subcore_axis_name="s",
                                  num_cores=_SC_NC, num_subcores=_SC_NS)

    def body(tblr_hbm, tblc_hbm, row_hbm, col_hbm, g_hbm, gd_hbm,
             ri_v, ci_v, gb_v, gdb_v, sem_r, sem_c):
        wid = jax.lax.axis_index("s") * _SC_NC + jax.lax.axis_index("c")
        base = wid * per_w
        for j in range(n_ch):
            off = base + j * ch
            pltpu.sync_copy(row_hbm.at[pl.ds(off, ch)], ri_v)
            pltpu.sync_copy(col_hbm.at[pl.ds(off, ch)], ci_v)
            a = pltpu.async_copy(tblr_hbm.at[ri_v], gb_v, sem_r)
            b = pltpu.async_copy(tblc_hbm.at[ci_v], gdb_v, sem_c)
            a.wait()
            b.wait()
            pltpu.sync_copy(gb_v, g_hbm.at[pl.ds(off, ch)])
            pltpu.sync_copy(gdb_v, gd_hbm.at[pl.ds(off, ch)])

    f = pl.kernel(
        body,
        out_type=[jax.ShapeDtypeStruct((e, dr), dtr),
                  jax.ShapeDtypeStruct((e, dc), dtc)],
        mesh=mesh,
        scratch_types=[
            pltpu.VMEM((ch,), jnp.int32),
            pltpu.VMEM((ch,), jnp.int32),
            pltpu.VMEM((ch, dr), dtr),
            pltpu.VMEM((ch, dc), dtc),
            pltpu.SemaphoreType.DMA,
            pltpu.SemaphoreType.DMA,
        ],
    )
    return f(tblr, tblc, row, col)


def _sc_scatter_add(vals, idx, n):
    e, w = vals.shape
    nw = _SC_NC * _SC_NS
    per_w = e // nw
    ch = _pick_ch(per_w)
    n_ch = per_w // ch
    n2 = ((n + 16 * 8 - 1) // (16 * 8)) * (16 * 8)
    rps = n2 // _SC_NS
    mesh = plsc.VectorSubcoreMesh(core_axis_name="c", subcore_axis_name="s",
                                  num_cores=_SC_NC, num_subcores=_SC_NS)

    def body(vals_hbm, idx_hbm, zeros_hbm, out_hbm, vb, iv, acc_sh):
        c = jax.lax.axis_index("c")
        s = jax.lax.axis_index("s")
        wid = s * _SC_NC + c
        pltpu.sync_copy(zeros_hbm.at[pl.ds(s * rps, rps)],
                        acc_sh.at[pl.ds(s * rps, rps)])
        plsc.subcore_barrier()
        base = wid * per_w
        for j in range(n_ch):
            off = base + j * ch
            pltpu.sync_copy(idx_hbm.at[pl.ds(off, ch)], iv)
            pltpu.sync_copy(vals_hbm.at[pl.ds(off, ch)], vb)
            pltpu.sync_copy(vb, acc_sh.at[iv], add=True)
        plsc.subcore_barrier()
        pltpu.sync_copy(acc_sh.at[pl.ds(s * rps, rps)],
                        out_hbm.at[pl.ds(c * n2 + s * rps, rps)])

    f = pl.kernel(
        body,
        out_type=jax.ShapeDtypeStruct((2 * n2, w), jnp.float32),
        mesh=mesh,
        scratch_types=[
            pltpu.VMEM((ch, w), jnp.float32),
            pltpu.VMEM((ch,), jnp.int32),
            pltpu.VMEM_SHARED((n2, w), jnp.float32),
        ],
    )
    out = f(vals, idx, jnp.zeros((n2, w), jnp.float32))
    return out[:n] + out[n2:n2 + n]


def _edge_l1_body(g_ref, gd_ref, eidx_ref, t2_ref, w2t_ref, b2_ref,
                  wn1et_ref, bn1_ref, wn2t_ref, bn2_ref, ea_ref, n1_ref):
    d = gd_ref.shape[1]
    g = g_ref[...].astype(jnp.float32)
    gd = gd_ref[...].astype(jnp.float32)
    eidx = eidx_ref[0, 0, :]
    sel = jnp.where(eidx[:, None] == 0, t2_ref[0:1, :], t2_ref[1:2, :])
    h1 = _relu(g[:, :d] + gd + sel)
    ea = h1 + _relu(_dot(h1, w2t_ref[...]) + b2_ref[...])
    n1h = _relu(g[:, d:] + _dot(ea, wn1et_ref[...]) + bn1_ref[...])
    n1 = n1h + _relu(_dot(n1h, wn2t_ref[...]) + bn2_ref[...])
    ea_ref[...] = ea
    n1_ref[...] = n1


def _edge_l2_body(g_ref, gd_ref, ea_ref, w1et_ref, b1_ref, w2t_ref, b2_ref,
                  wn1et_ref, bn1_ref, wn2t_ref, bn2_ref, n1_ref):
    d = gd_ref.shape[1]
    g = g_ref[...].astype(jnp.float32)
    gd = gd_ref[...].astype(jnp.float32)
    h1 = _relu(g[:, :d] + gd
               + _dot(ea_ref[...], w1et_ref[...]) + b1_ref[...])
    ea = h1 + _relu(_dot(h1, w2t_ref[...]) + b2_ref[...])
    n1h = _relu(g[:, d:] + _dot(ea, wn1et_ref[...]) + bn1_ref[...])
    n1 = n1h + _relu(_dot(n1h, wn2t_ref[...]) + bn2_ref[...])
    n1_ref[...] = n1


def _edge_block_spec(be, d):
    return pl.BlockSpec((be, d), lambda i: (i, 0))


def _const_spec(shape):
    nd = len(shape)
    return pl.BlockSpec(shape, lambda i: (0,) * nd)


def _run_edge_l1(g, gd, eidx3, t2, w2t, b2, wn1et, bn1, wn2t, bn2, be):
    ep, d = gd.shape
    grid = ep // be
    out_shape = [jax.ShapeDtypeStruct((ep, d), jnp.float32),
                 jax.ShapeDtypeStruct((ep, d), jnp.float32)]
    return pl.pallas_call(
        _edge_l1_body,
        grid=(grid,),
        in_specs=[
            _edge_block_spec(be, 2 * d),
            _edge_block_spec(be, d),
            pl.BlockSpec((1, 1, be), lambda i: (i, 0, 0)),
            _const_spec((2, d)),
            _const_spec((d, d)), _const_spec((1, d)),
            _const_spec((d, d)), _const_spec((1, d)),
            _const_spec((d, d)), _const_spec((1, d)),
        ],
        out_specs=[_edge_block_spec(be, d), _edge_block_spec(be, d)],
        out_shape=out_shape,
        compiler_params=pltpu.CompilerParams(
            dimension_semantics=("arbitrary",)),
    )(g, gd, eidx3, t2, w2t, b2, wn1et, bn1, wn2t, bn2)


def _run_edge_l2(g, gd, ea, w1et, b1, w2t, b2, wn1et, bn1, wn2t, bn2, be):
    ep, d = gd.shape
    grid = ep // be
    return pl.pallas_call(
        _edge_l2_body,
        grid=(grid,),
        in_specs=[
            _edge_block_spec(be, 2 * d),
            _edge_block_spec(be, d),
            _edge_block_spec(be, d),
            _const_spec((d, d)), _const_spec((1, d)),
            _const_spec((d, d)), _const_spec((1, d)),
            _const_spec((d, d)), _const_spec((1, d)),
            _const_spec((d, d)), _const_spec((1, d)),
        ],
        out_specs=[_edge_block_spec(be, d)],
        out_shape=[jax.ShapeDtypeStruct((ep, d), jnp.float32)],
        compiler_params=pltpu.CompilerParams(
            dimension_semantics=("arbitrary",)),
    )(g, gd, ea, w1et, b1, w2t, b2, wn1et, bn1, wn2t, bn2)[0]


def kernel(x_fp, x_feature, params, centers, edge_index, edge_attr_idx,
           batch, num_graphs):
    n = x_fp.shape[0]
    e = edge_index.shape[1]
    d = params["edge_embed"].shape[1]

    mol = x_feature[:, 0] == 0.0
    rec = x_feature[:, 0] == 1.0
    fp = _dot(x_fp, params["fp_W"].T)
    r3 = _rbf(x_feature[:, 3], centers)
    r4 = _rbf(x_feature[:, 4], centers)
    first = jnp.where(mol[:, None], fp, jnp.where(rec[:, None], r3, 0.0))
    second = jnp.where((mol | rec)[:, None], r4, 0.0)
    x = jnp.concatenate([first, second], axis=1)
    u = jnp.zeros((BSZ, d), jnp.float32) + jnp.asarray(num_graphs, jnp.float32) * 0.0

    row = edge_index[0]
    col = edge_index[1]

    be = min(_BE, e)
    ep = ((e + be - 1) // be) * be
    pad = ep - e
    row_p = jnp.concatenate([row, jnp.zeros((pad,), row.dtype)]) if pad else row
    col_scat = jnp.concatenate([col, jnp.full((pad,), n, col.dtype)]) if pad else col
    col_p = jnp.concatenate([col, jnp.zeros((pad,), col.dtype)]) if pad else col
    eidx_p = (jnp.concatenate([edge_attr_idx, jnp.zeros((pad,), edge_attr_idx.dtype)])
              if pad else edge_attr_idx)
    eidx3 = eidx_p.astype(jnp.int32).reshape(ep // be, 1, be)

    use_sc = (pad == 0) and e % (_SC_NC * _SC_NS * _SC_CH) == 0


    deg = jax.ops.segment_sum(jnp.ones((ep,), jnp.float32), col_scat,
                              num_segments=n + 1)[:n]
    deg = jnp.maximum(deg, 1.0)[:, None]

    onehot_g = (batch[None, :] == jnp.arange(BSZ, dtype=batch.dtype)[:, None]
                ).astype(jnp.float32)
    gcount = jnp.maximum(jnp.sum(onehot_g, axis=1), 1.0)[:, None]

    for li, lp in enumerate(params["layers"]):
        w1 = lp["edge"][0]["W"]
        b1 = lp["edge"][0]["b"]
        w2 = lp["edge"][1]["W"]
        b2 = lp["edge"][1]["b"]
        wn1 = lp["node1"][0]["W"]
        bn1 = lp["node1"][0]["b"]
        wn2 = lp["node1"][1]["W"]
        bn2 = lp["node1"][1]["b"]

        psu = _dot(x, w1[:, :d].T) + _dot(u, w1[:, 3 * d:].T)[batch]
        pd = _dot(x, w1[:, d:2 * d].T)
        px = _dot(x, wn1[:, :d].T)

        t2 = _dot(params["edge_embed"], w1[:, 2 * d:3 * d].T) + b1[None, :]
        wargs_l1 = (w2.T, b2[None, :], wn1[:, d:].T, bn1[None, :],
                    wn2.T, bn2[None, :])
        wargs_l2 = (w1[:, 2 * d:3 * d].T, b1[None, :]) + wargs_l1

        if use_sc:
            tblr = jnp.concatenate([psu, px], axis=1)
            g, gd = _sc_gather_pair(tblr, pd, row, col)
            if li == 0:
                ea, n1 = _run_edge_l1(g, gd, eidx3, t2, *wargs_l1, be)
            else:
                n1 = _run_edge_l2(g, gd, ea, *wargs_l2, be)
            agg = _sc_scatter_add(n1, col, n) / deg
        else:
            g = jnp.concatenate([psu[row_p], px[row_p]], axis=1)
            gd = pd[col_p]
            if li == 0:
                ea, n1 = _run_edge_l1(g, gd, eidx3, t2, *wargs_l1, be)
            else:
                n1 = _run_edge_l2(g, gd, ea, *wargs_l2, be)
            agg = jax.ops.segment_sum(n1, col_scat, num_segments=n + 1)[:n] / deg

        wa = lp["node2"][0]["W"]
        ba = lp["node2"][0]["b"]
        wb = lp["node2"][1]["W"]
        bb = lp["node2"][1]["b"]
        xh = _relu(_dot(x, wa[:, :d].T) + _dot(agg, wa[:, d:2 * d].T)
                   + _dot(u, wa[:, 2 * d:].T)[batch] + ba[None, :])
        x = xh + _relu(_dot(xh, wb.T) + bb[None, :])

        if li + 1 < len(params["layers"]):
            gmean = _dot(onehot_g, x) / gcount
            wg = lp["global"][0]["W"]
            bg = lp["global"][0]["b"]
            wg2 = lp["global"][1]["W"]
            bg2 = lp["global"][1]["b"]
            uh = _relu(_dot(u, wg[:, :d].T) + _dot(gmean, wg[:, d:].T)
                       + bg[None, :])
            u = uh + _relu(_dot(uh, wg2.T) + bg2[None, :])

    return _dot(x, params["out"]["W"].T) + params["out"]["b"][None, :]


# speedup vs baseline: 2.2875x; 1.0546x over previous
import functools

import jax
import jax.numpy as jnp
from jax.experimental import pallas as pl
from jax.experimental.pallas import tpu as pltpu
from jax.experimental.pallas import tpu_sc as plsc

BSZ = 16
_BE = 2000
_SC_NC = 2
_SC_NS = 16
_SC_CH = 200


def _pick_ch(per_w):
    for ch in (256, 200, 128, 104, 64, 40, 8):
        if per_w % ch == 0:
            return ch
    return None


def _rbf(v, centers):
    gammap = ((10.0 - 0.0) / 2.0) ** 2
    return jnp.exp(-((v[:, None] - centers[None, :]) ** 2) / gammap)


def _relu(v):
    return jnp.maximum(v, 0.0)


def _dot(a, b):
    return jnp.dot(a, b, preferred_element_type=jnp.float32)


def _sc_gather_pair(tblr, tblc, row, col):
    e = row.shape[0]
    nw = _SC_NC * _SC_NS
    per_w = e // nw
    ch = _pick_ch(per_w)
    n_ch = per_w // ch
    dr = tblr.shape[1]
    dc = tblc.shape[1]
    dtr = tblr.dtype
    dtc = tblc.dtype
    mesh = plsc.VectorSubcoreMesh(core_axis_name="c", subcore_axis_name="s",
                                  num_cores=_SC_NC, num_subcores=_SC_NS)

    def body(tblr_hbm, tblc_hbm, row_hbm, col_hbm, g_hbm, gd_hbm,
             ri_v, ci_v, gb_v, gdb_v, sem_r, sem_c, sem_sr, sem_sc):
        wid = jax.lax.axis_index("s") * _SC_NC + jax.lax.axis_index("c")
        base = wid * per_w
        pltpu.sync_copy(row_hbm.at[pl.ds(base, per_w)], ri_v)
        pltpu.sync_copy(col_hbm.at[pl.ds(base, per_w)], ci_v)
        sa = sb = None
        for j in range(n_ch):
            off = base + j * ch
            if sa is not None:
                sa.wait()
                sb.wait()
            a = pltpu.async_copy(tblr_hbm.at[ri_v.at[pl.ds(j * ch, ch)]],
                                 gb_v, sem_r)
            b = pltpu.async_copy(tblc_hbm.at[ci_v.at[pl.ds(j * ch, ch)]],
                                 gdb_v, sem_c)
            a.wait()
            b.wait()
            sa = pltpu.async_copy(gb_v, g_hbm.at[pl.ds(off, ch)], sem_sr)
            sb = pltpu.async_copy(gdb_v, gd_hbm.at[pl.ds(off, ch)], sem_sc)
        sa.wait()
        sb.wait()

    f = pl.kernel(
        body,
        out_type=[jax.ShapeDtypeStruct((e, dr), dtr),
                  jax.ShapeDtypeStruct((e, dc), dtc)],
        mesh=mesh,
        scratch_types=[
            pltpu.VMEM((per_w,), jnp.int32),
            pltpu.VMEM((per_w,), jnp.int32),
            pltpu.VMEM((ch, dr), dtr),
            pltpu.VMEM((ch, dc), dtc),
            pltpu.SemaphoreType.DMA,
            pltpu.SemaphoreType.DMA,
            pltpu.SemaphoreType.DMA,
            pltpu.SemaphoreType.DMA,
        ],
    )
    return f(tblr, tblc, row, col)


def _sc_scatter_add(vals, idx, n):
    e, w = vals.shape
    nw = _SC_NC * _SC_NS
    per_w = e // nw
    ch = _pick_ch(per_w)
    n_ch = per_w // ch
    n2 = ((n + 16 * 8 - 1) // (16 * 8)) * (16 * 8)
    rps = n2 // _SC_NS
    mesh = plsc.VectorSubcoreMesh(core_axis_name="c", subcore_axis_name="s",
                                  num_cores=_SC_NC, num_subcores=_SC_NS)

    def body(vals_hbm, idx_hbm, zeros_hbm, out_hbm, vb, iv, acc_sh):
        c = jax.lax.axis_index("c")
        s = jax.lax.axis_index("s")
        wid = s * _SC_NC + c
        base = wid * per_w
        pltpu.sync_copy(zeros_hbm.at[pl.ds(s * rps, rps)],
                        acc_sh.at[pl.ds(s * rps, rps)])
        plsc.subcore_barrier()
        for j in range(n_ch):
            off = base + j * ch
            pltpu.sync_copy(idx_hbm.at[pl.ds(off, ch)], iv)
            pltpu.sync_copy(vals_hbm.at[pl.ds(off, ch)], vb)
            pltpu.sync_copy(vb, acc_sh.at[iv], add=True)
        plsc.subcore_barrier()
        pltpu.sync_copy(acc_sh.at[pl.ds(s * rps, rps)],
                        out_hbm.at[pl.ds(c * n2 + s * rps, rps)])

    f = pl.kernel(
        body,
        out_type=jax.ShapeDtypeStruct((2 * n2, w), jnp.float32),
        mesh=mesh,
        scratch_types=[
            pltpu.VMEM((ch, w), jnp.float32),
            pltpu.VMEM((ch,), jnp.int32),
            pltpu.VMEM_SHARED((n2, w), jnp.float32),
        ],
    )
    out = f(vals, idx, jnp.zeros((n2, w), jnp.float32))
    return out[:n] + out[n2:n2 + n]


def _edge_l1_body(g_ref, gd_ref, eidx_ref, t2_ref, w2t_ref, b2_ref,
                  wn1et_ref, bn1_ref, wn2t_ref, bn2_ref, ea_ref, n1_ref):
    d = gd_ref.shape[1]
    g = g_ref[...].astype(jnp.float32)
    gd = gd_ref[...].astype(jnp.float32)
    eidx = eidx_ref[0, 0, :]
    sel = jnp.where(eidx[:, None] == 0, t2_ref[0:1, :], t2_ref[1:2, :])
    h1 = _relu(g[:, :d] + gd + sel)
    ea = h1 + _relu(_dot(h1, w2t_ref[...]) + b2_ref[...])
    n1h = _relu(g[:, d:] + _dot(ea, wn1et_ref[...]) + bn1_ref[...])
    n1 = n1h + _relu(_dot(n1h, wn2t_ref[...]) + bn2_ref[...])
    ea_ref[...] = ea
    n1_ref[...] = n1


def _edge_l2_body(g_ref, gd_ref, ea_ref, w1et_ref, b1_ref, w2t_ref, b2_ref,
                  wn1et_ref, bn1_ref, wn2t_ref, bn2_ref, n1_ref):
    d = gd_ref.shape[1]
    g = g_ref[...].astype(jnp.float32)
    gd = gd_ref[...].astype(jnp.float32)
    h1 = _relu(g[:, :d] + gd
               + _dot(ea_ref[...], w1et_ref[...]) + b1_ref[...])
    ea = h1 + _relu(_dot(h1, w2t_ref[...]) + b2_ref[...])
    n1h = _relu(g[:, d:] + _dot(ea, wn1et_ref[...]) + bn1_ref[...])
    n1 = n1h + _relu(_dot(n1h, wn2t_ref[...]) + bn2_ref[...])
    n1_ref[...] = n1


def _edge_block_spec(be, d):
    return pl.BlockSpec((be, d), lambda i: (i, 0))


def _const_spec(shape):
    nd = len(shape)
    return pl.BlockSpec(shape, lambda i: (0,) * nd)


def _run_edge_l1(g, gd, eidx3, t2, w2t, b2, wn1et, bn1, wn2t, bn2, be):
    ep, d = gd.shape
    grid = ep // be
    out_shape = [jax.ShapeDtypeStruct((ep, d), jnp.float32),
                 jax.ShapeDtypeStruct((ep, d), jnp.float32)]
    return pl.pallas_call(
        _edge_l1_body,
        grid=(grid,),
        in_specs=[
            _edge_block_spec(be, 2 * d),
            _edge_block_spec(be, d),
            pl.BlockSpec((1, 1, be), lambda i: (i, 0, 0)),
            _const_spec((2, d)),
            _const_spec((d, d)), _const_spec((1, d)),
            _const_spec((d, d)), _const_spec((1, d)),
            _const_spec((d, d)), _const_spec((1, d)),
        ],
        out_specs=[_edge_block_spec(be, d), _edge_block_spec(be, d)],
        out_shape=out_shape,
        compiler_params=pltpu.CompilerParams(
            dimension_semantics=("arbitrary",)),
    )(g, gd, eidx3, t2, w2t, b2, wn1et, bn1, wn2t, bn2)


def _run_edge_l2(g, gd, ea, w1et, b1, w2t, b2, wn1et, bn1, wn2t, bn2, be):
    ep, d = gd.shape
    grid = ep // be
    return pl.pallas_call(
        _edge_l2_body,
        grid=(grid,),
        in_specs=[
            _edge_block_spec(be, 2 * d),
            _edge_block_spec(be, d),
            _edge_block_spec(be, d),
            _const_spec((d, d)), _const_spec((1, d)),
            _const_spec((d, d)), _const_spec((1, d)),
            _const_spec((d, d)), _const_spec((1, d)),
            _const_spec((d, d)), _const_spec((1, d)),
        ],
        out_specs=[_edge_block_spec(be, d)],
        out_shape=[jax.ShapeDtypeStruct((ep, d), jnp.float32)],
        compiler_params=pltpu.CompilerParams(
            dimension_semantics=("arbitrary",)),
    )(g, gd, ea, w1et, b1, w2t, b2, wn1et, bn1, wn2t, bn2)[0]


def kernel(x_fp, x_feature, params, centers, edge_index, edge_attr_idx,
           batch, num_graphs):
    n = x_fp.shape[0]
    e = edge_index.shape[1]
    d = params["edge_embed"].shape[1]

    mol = x_feature[:, 0] == 0.0
    rec = x_feature[:, 0] == 1.0
    fp = _dot(x_fp, params["fp_W"].T)
    r3 = _rbf(x_feature[:, 3], centers)
    r4 = _rbf(x_feature[:, 4], centers)
    first = jnp.where(mol[:, None], fp, jnp.where(rec[:, None], r3, 0.0))
    second = jnp.where((mol | rec)[:, None], r4, 0.0)
    x = jnp.concatenate([first, second], axis=1)
    u = jnp.zeros((BSZ, d), jnp.float32)

    row = edge_index[0]
    col = edge_index[1]

    be = min(_BE, e)
    ep = ((e + be - 1) // be) * be
    pad = ep - e
    row_p = jnp.concatenate([row, jnp.zeros((pad,), row.dtype)]) if pad else row
    col_scat = jnp.concatenate([col, jnp.full((pad,), n, col.dtype)]) if pad else col
    col_p = jnp.concatenate([col, jnp.zeros((pad,), col.dtype)]) if pad else col
    eidx_p = (jnp.concatenate([edge_attr_idx, jnp.zeros((pad,), edge_attr_idx.dtype)])
              if pad else edge_attr_idx)
    eidx3 = eidx_p.astype(jnp.int32).reshape(ep // be, 1, be)

    use_sc = (pad == 0) and e % (_SC_NC * _SC_NS * _SC_CH) == 0


    deg = jax.ops.segment_sum(jnp.ones((ep,), jnp.float32), col_scat,
                              num_segments=n + 1)[:n]
    deg = jnp.maximum(deg, 1.0)[:, None]

    onehot_g = (batch[None, :] == jnp.arange(BSZ, dtype=batch.dtype)[:, None]
                ).astype(jnp.float32)
    gcount = jnp.maximum(jnp.sum(onehot_g, axis=1), 1.0)[:, None]

    for li, lp in enumerate(params["layers"]):
        w1 = lp["edge"][0]["W"]
        b1 = lp["edge"][0]["b"]
        w2 = lp["edge"][1]["W"]
        b2 = lp["edge"][1]["b"]
        wn1 = lp["node1"][0]["W"]
        bn1 = lp["node1"][0]["b"]
        wn2 = lp["node1"][1]["W"]
        bn2 = lp["node1"][1]["b"]

        psu = _dot(x, w1[:, :d].T) + _dot(u, w1[:, 3 * d:].T)[batch]
        pd = _dot(x, w1[:, d:2 * d].T)
        px = _dot(x, wn1[:, :d].T)

        t2 = _dot(params["edge_embed"], w1[:, 2 * d:3 * d].T) + b1[None, :]
        wargs_l1 = (w2.T, b2[None, :], wn1[:, d:].T, bn1[None, :],
                    wn2.T, bn2[None, :])
        wargs_l2 = (w1[:, 2 * d:3 * d].T, b1[None, :]) + wargs_l1

        if use_sc:
            tblr = jnp.concatenate([psu, px], axis=1)
            g, gd = _sc_gather_pair(tblr, pd, row, col)
            if li == 0:
                ea, n1 = _run_edge_l1(g, gd, eidx3, t2, *wargs_l1, be)
            else:
                n1 = _run_edge_l2(g, gd, ea, *wargs_l2, be)
            agg = _sc_scatter_add(n1, col, n) / deg
        else:
            g = jnp.concatenate([psu[row_p], px[row_p]], axis=1)
            gd = pd[col_p]
            if li == 0:
                ea, n1 = _run_edge_l1(g, gd, eidx3, t2, *wargs_l1, be)
            else:
                n1 = _run_edge_l2(g, gd, ea, *wargs_l2, be)
            agg = jax.ops.segment_sum(n1, col_scat, num_segments=n + 1)[:n] / deg

        wa = lp["node2"][0]["W"]
        ba = lp["node2"][0]["b"]
        wb = lp["node2"][1]["W"]
        bb = lp["node2"][1]["b"]
        xh = _relu(_dot(x, wa[:, :d].T) + _dot(agg, wa[:, d:2 * d].T)
                   + _dot(u, wa[:, 2 * d:].T)[batch] + ba[None, :])
        x = xh + _relu(_dot(xh, wb.T) + bb[None, :])

        if li + 1 < len(params["layers"]):
            gmean = _dot(onehot_g, x) / gcount
            wg = lp["global"][0]["W"]
            bg = lp["global"][0]["b"]
            wg2 = lp["global"][1]["W"]
            bg2 = lp["global"][1]["b"]
            uh = _relu(_dot(u, wg[:, :d].T) + _dot(gmean, wg[:, d:].T)
                       + bg[None, :])
            u = uh + _relu(_dot(uh, wg2.T) + bg2[None, :])

    return _dot(x, params["out"]["W"].T) + params["out"]["b"][None, :]
